# Initial kernel scaffold; baseline (speedup 1.0000x reference)
import jax
import jax.numpy as jnp
from jax.experimental import pallas as pl


def kernel(items, user_h_0, user_r_0, user_t_0, user_h_1, user_r_1, user_t_1, item_h_0, item_r_0, item_t_0, item_h_1, item_r_1, item_t_1, entity_emb, relation_emb, W1, W2, W3):
    raise NotImplementedError("write your pallas kernel here")


# speedup vs baseline: 1.8556x; 1.8556x over previous
import functools

import jax
import jax.numpy as jnp
from jax import lax
from jax.experimental import pallas as pl
from jax.experimental.pallas import tpu as pltpu
from jax.experimental.pallas import tpu_sc as plsc

NC, NS = 2, 16
NW = NC * NS
B, T, D, NREL = 4096, 32, 64, 200
NSETS = 4
N = NSETS * T * B
NROWS = N + B
RPW = NROWS // NW
GCH = 3
GIT = RPW // (GCH * 128)

BB = 512
NB = B // BB

CB = 4
NCH = (B // NW) // CB


def _sc_mesh():
    return plsc.VectorSubcoreMesh(core_axis_name="c", subcore_axis_name="s")


def _wid():
    return lax.axis_index("s") * NC + lax.axis_index("c")


@functools.partial(
    pl.kernel,
    out_type=jax.ShapeDtypeStruct((NROWS, D), jnp.float32),
    mesh=_sc_mesh(),
    compiler_params=pltpu.CompilerParams(use_tc_tiling_on_sc=False, needs_layout_passes=False),
    scratch_types=[
        pltpu.VMEM((GCH * 128,), jnp.int32),
        pltpu.VMEM((GCH * 128, D), jnp.float32),
        pltpu.SemaphoreType.DMA,
    ],
)
def _gather_rows(emb, idxh, out, idx_v, rows_v, sem):
    w = _wid()
    base = w * RPW

    def step(g, carry):
        r0 = base + GCH * 128 * g
        pltpu.sync_copy(idxh.at[pl.ds(r0, GCH * 128)], idx_v)
        cps = [
            pltpu.async_copy(emb.at[idx_v.at[pl.ds(128 * k, 128)]],
                             rows_v.at[pl.ds(128 * k, 128)], sem)
            for k in range(GCH)
        ]
        for c in cps:
            c.wait()
        pltpu.sync_copy(rows_v, out.at[pl.ds(r0, GCH * 128)])
        return carry

    lax.fori_loop(0, GIT, step, 0)


def _att_body(g_ref, r_ref, rel_ref, w1_ref, w2_ref, w3_ref,
              z_ref, uhm_ref, ihm_ref, r1_ref):
    j = pl.program_id(1)

    @pl.when(j == 0)
    def _():
        r1_ref[...] = jnp.dot(rel_ref[...], w1_ref[D:2 * D, :],
                              preferred_element_type=jnp.float32,
                              precision=lax.Precision.HIGHEST)

    h = g_ref[...]
    rv = r_ref[...].astype(jnp.int32)
    iot = lax.broadcasted_iota(jnp.int32, (BB, NREL), 1)
    oh = (rv == iot).astype(jnp.float32)
    x = jnp.dot(h, w1_ref[0:D, :], preferred_element_type=jnp.float32,
                precision=lax.Precision.HIGHEST)
    x = x + jnp.dot(oh, r1_ref[...], preferred_element_type=jnp.float32,
                    precision=lax.Precision.HIGHEST)
    x = jnp.maximum(x, 0.0)
    x = jnp.maximum(jnp.dot(x, w2_ref[...], preferred_element_type=jnp.float32,
                            precision=lax.Precision.HIGHEST), 0.0)
    z_ref[...] = jnp.dot(x, w3_ref[...], preferred_element_type=jnp.float32,
                         precision=lax.Precision.HIGHEST)

    inv = 1.0 / T

    @pl.when(j == 0)
    def _():
        uhm_ref[...] = h * inv

    @pl.when((j >= 1) & (j < T))
    def _():
        uhm_ref[...] = uhm_ref[...] + h * inv

    @pl.when(j == 2 * T)
    def _():
        ihm_ref[...] = h * inv

    @pl.when((j > 2 * T) & (j < 3 * T))
    def _():
        ihm_ref[...] = ihm_ref[...] + h * inv


def _att_call(grows, r_col, rel, w1, w2, w3):
    return pl.pallas_call(
        _att_body,
        grid=(NB, NSETS * T),
        in_specs=[
            pl.BlockSpec((BB, D), lambda i, j: (j * NB + i, 0)),
            pl.BlockSpec((BB, 1), lambda i, j: (j * NB + i, 0)),
            pl.BlockSpec((NREL, D), lambda i, j: (0, 0)),
            pl.BlockSpec((2 * D, D), lambda i, j: (0, 0)),
            pl.BlockSpec((D, D), lambda i, j: (0, 0)),
            pl.BlockSpec((D, 1), lambda i, j: (0, 0)),
        ],
        out_specs=[
            pl.BlockSpec((BB, 1), lambda i, j: (j * NB + i, 0)),
            pl.BlockSpec((BB, D), lambda i, j: (i, 0)),
            pl.BlockSpec((BB, D), lambda i, j: (i, 0)),
        ],
        out_shape=[
            jax.ShapeDtypeStruct((N, 1), jnp.float32),
            jax.ShapeDtypeStruct((B, D), jnp.float32),
            jax.ShapeDtypeStruct((B, D), jnp.float32),
        ],
        scratch_shapes=[pltpu.VMEM((NREL, D), jnp.float32)],
    )(grows, r_col, rel, w1, w2, w3)


@functools.partial(
    pl.kernel,
    out_type=jax.ShapeDtypeStruct((B,), jnp.float32),
    mesh=_sc_mesh(),
    compiler_params=pltpu.CompilerParams(use_tc_tiling_on_sc=False, needs_layout_passes=False),
    scratch_types=[
        pltpu.VMEM((NSETS * 128,), jnp.int32),
        pltpu.VMEM((NSETS * CB * T,), jnp.float32),
        pltpu.VMEM((NSETS * CB * T, D), jnp.float32),
        pltpu.VMEM((CB * D,), jnp.float32),
        pltpu.VMEM((CB * D,), jnp.float32),
        pltpu.VMEM((CB * D,), jnp.float32),
        pltpu.VMEM((16,), jnp.float32),
        pltpu.SemaphoreType.DMA,
    ],
)
def _finisher(emb, tix, zb, uhm, ihm, grows, out,
              idx_v, zv, rows_v, uv, iv, gv, svec_ref, sem):
    w = _wid()
    bpw = B // NW

    def chunk(ch, svec):
        b0 = w * bpw + ch * CB
        for s in range(NSETS):
            pltpu.sync_copy(tix.at[pl.ds(s * B * T + b0 * T, CB * T)],
                            idx_v.at[pl.ds(s * 128, 128)])
            pltpu.sync_copy(zb.at[pl.ds(s * B * T + b0 * T, CB * T)],
                            zv.at[pl.ds(s * CB * T, CB * T)])
        pltpu.sync_copy(uhm.at[pl.ds(b0 * D, CB * D)], uv)
        pltpu.sync_copy(ihm.at[pl.ds(b0 * D, CB * D)], iv)
        pltpu.sync_copy(grows.at[pl.ds((N + b0) * D, CB * D)], gv)
        cps = [
            pltpu.async_copy(emb.at[idx_v.at[pl.ds(128 * k, 128)]],
                             rows_v.at[pl.ds(128 * k, 128)], sem)
            for k in range(NSETS)
        ]
        for c in cps:
            c.wait()
        for bl in range(CB):
            eu = [uv[pl.ds(bl * D + 16 * k, 16)] for k in range(4)]
            ev = [iv[pl.ds(bl * D + 16 * k, 16)] + gv[pl.ds(bl * D + 16 * k, 16)]
                  for k in range(4)]
            for s in range(NSETS):
                zo = s * CB * T + bl * T
                v0 = zv[pl.ds(zo, 16)]
                v1 = zv[pl.ds(zo + 16, 16)]
                e0 = jnp.exp(1.0 / (1.0 + jnp.exp(-v0)))
                e1 = jnp.exp(1.0 / (1.0 + jnp.exp(-v1)))
                tot = jnp.sum(e0 + e1)
                w0 = e0 / tot
                w1 = e1 / tot
                acc = [jnp.zeros((16,), jnp.float32) for _ in range(4)]
                for t in range(T):
                    wt = w0[t] if t < 16 else w1[t - 16]
                    for k in range(4):
                        acc[k] = acc[k] + wt * rows_v[zo + t, pl.ds(16 * k, 16)]
                if s < 2:
                    eu = [eu[k] + acc[k] for k in range(4)]
                else:
                    ev = [ev[k] + acc[k] for k in range(4)]
            dotv = eu[0] * ev[0]
            for k in range(1, 4):
                dotv = dotv + eu[k] * ev[k]
            dot = jnp.sum(dotv)
            scv = 1.0 / (1.0 + jnp.exp(-jnp.full((16,), dot, jnp.float32)))
            lane = (ch % 4) * CB + bl
            svec = jnp.where(lax.iota(jnp.int32, 16) == lane, scv, svec)
        svec_ref[...] = svec
        pltpu.sync_copy(svec_ref,
                        out.at[pl.ds(w * bpw + (ch // 4) * 16, 16)])
        return svec

    lax.fori_loop(0, NCH, chunk, jnp.zeros((16,), jnp.float32))


def kernel(items, user_h_0, user_r_0, user_t_0, user_h_1, user_r_1, user_t_1,
           item_h_0, item_r_0, item_t_0, item_h_1, item_r_1, item_t_1,
           entity_emb, relation_emb, W1, W2, W3):
    h_idx = jnp.concatenate([
        user_h_0.T.reshape(-1), user_h_1.T.reshape(-1),
        item_h_0.T.reshape(-1), item_h_1.T.reshape(-1),
        items.reshape(-1),
    ]).astype(jnp.int32)
    grows = _gather_rows(entity_emb, h_idx)

    r_col = jnp.stack([user_r_0.T, user_r_1.T, item_r_0.T, item_r_1.T]
                      ).astype(jnp.float32).reshape(N, 1)
    z, uhm, ihm = _att_call(grows, r_col, relation_emb, W1, W2, W3)

    zb = z.reshape(NSETS, T, B).transpose(0, 2, 1).reshape(NSETS * B * T)
    tix = jnp.stack([user_t_0, user_t_1, item_t_0, item_t_1]
                    ).astype(jnp.int32).reshape(NSETS * B * T)
    return _finisher(entity_emb, tix, zb, uhm.reshape(B * D),
                     ihm.reshape(B * D), grows.reshape(NROWS * D))


# speedup vs baseline: 2.9321x; 1.5802x over previous
import functools

import jax
import jax.numpy as jnp
from jax import lax
from jax.experimental import pallas as pl
from jax.experimental.pallas import tpu as pltpu
from jax.experimental.pallas import tpu_sc as plsc

NC, NS = 2, 16
NW = NC * NS
B, T, D, NREL = 4096, 32, 64, 200
NSETS = 4
N = NSETS * T * B
RPW = N // NW
GCH = 4
GIT = RPW // (GCH * 128)

BB = 1024
NB = B // BB

CB = 4
NCH = (B // NW) // CB

_SC_PARAMS = dict(
    mesh=plsc.VectorSubcoreMesh(core_axis_name="c", subcore_axis_name="s"),
    compiler_params=pltpu.CompilerParams(
        use_tc_tiling_on_sc=False, needs_layout_passes=False),
)


def _wid():
    return lax.axis_index("s") * NC + lax.axis_index("c")


def _r1_body(rel_ref, w1_ref, out_ref):
    out_ref[...] = jnp.dot(rel_ref[...], w1_ref[D:2 * D, :],
                           preferred_element_type=jnp.float32,
                           precision=lax.Precision.HIGHEST)


def _r1_call(rel, w1):
    return pl.pallas_call(
        _r1_body,
        out_shape=jax.ShapeDtypeStruct((NREL, D), jnp.float32),
    )(rel, w1)


@functools.partial(
    pl.kernel,
    out_type=(jax.ShapeDtypeStruct((N, D), jnp.float32),
              jax.ShapeDtypeStruct((N, D), jnp.float32)),
    scratch_types=[
        pltpu.VMEM((GCH * 128,), jnp.int32),
        pltpu.VMEM((GCH * 128,), jnp.int32),
        pltpu.VMEM((GCH * 128, D), jnp.float32),
        pltpu.VMEM((GCH * 128, D), jnp.float32),
        pltpu.SemaphoreType.DMA,
    ],
    **_SC_PARAMS,
)
def _gather_rows(emb, r1t, hidx, ridx, outh, outr, hx_v, rx_v, hr_v, rr_v, sem):
    w = _wid()
    base = w * RPW

    def step(g, carry):
        r0 = base + GCH * 128 * g
        pltpu.sync_copy(hidx.at[pl.ds(r0, GCH * 128)], hx_v)
        pltpu.sync_copy(ridx.at[pl.ds(r0, GCH * 128)], rx_v)
        cps = []
        for k in range(GCH):
            cps.append(pltpu.async_copy(emb.at[hx_v.at[pl.ds(128 * k, 128)]],
                                        hr_v.at[pl.ds(128 * k, 128)], sem))
            cps.append(pltpu.async_copy(r1t.at[rx_v.at[pl.ds(128 * k, 128)]],
                                        rr_v.at[pl.ds(128 * k, 128)], sem))
        for c in cps:
            c.wait()
        pltpu.sync_copy(hr_v, outh.at[pl.ds(r0, GCH * 128)])
        pltpu.sync_copy(rr_v, outr.at[pl.ds(r0, GCH * 128)])
        return carry

    lax.fori_loop(0, GIT, step, 0)


def _att_body(gh_ref, gr_ref, w1_ref, w2_ref, w3_ref,
              z_ref, uhm_ref, ihm_ref):
    j = pl.program_id(1)

    h = gh_ref[...]
    x = jnp.dot(h.astype(jnp.bfloat16),
                w1_ref[0:D, :].astype(jnp.bfloat16),
                preferred_element_type=jnp.float32)
    x = jnp.maximum(x + gr_ref[...], 0.0)
    x = jnp.maximum(jnp.dot(x.astype(jnp.bfloat16),
                            w2_ref[...].astype(jnp.bfloat16),
                            preferred_element_type=jnp.float32), 0.0)
    z_ref[...] = jnp.dot(x.astype(jnp.bfloat16),
                         w3_ref[...].astype(jnp.bfloat16),
                         preferred_element_type=jnp.float32)

    inv = 1.0 / T

    @pl.when(j == 0)
    def _():
        uhm_ref[...] = h * inv

    @pl.when((j >= 1) & (j < T))
    def _():
        uhm_ref[...] = uhm_ref[...] + h * inv

    @pl.when(j == 2 * T)
    def _():
        ihm_ref[...] = h * inv

    @pl.when((j > 2 * T) & (j < 3 * T))
    def _():
        ihm_ref[...] = ihm_ref[...] + h * inv


def _att_call(gh, gr, w1, w2, w3):
    return pl.pallas_call(
        _att_body,
        grid=(NB, NSETS * T),
        in_specs=[
            pl.BlockSpec((BB, D), lambda i, j: (j * NB + i, 0)),
            pl.BlockSpec((BB, D), lambda i, j: (j * NB + i, 0)),
            pl.BlockSpec((2 * D, D), lambda i, j: (0, 0)),
            pl.BlockSpec((D, D), lambda i, j: (0, 0)),
            pl.BlockSpec((D, 1), lambda i, j: (0, 0)),
        ],
        out_specs=[
            pl.BlockSpec((BB, 1), lambda i, j: (j * NB + i, 0)),
            pl.BlockSpec((BB, D), lambda i, j: (i, 0)),
            pl.BlockSpec((BB, D), lambda i, j: (i, 0)),
        ],
        out_shape=[
            jax.ShapeDtypeStruct((N, 1), jnp.float32),
            jax.ShapeDtypeStruct((B, D), jnp.float32),
            jax.ShapeDtypeStruct((B, D), jnp.float32),
        ],
    )(gh, gr, w1, w2, w3)


@functools.partial(
    pl.kernel,
    out_type=jax.ShapeDtypeStruct((B,), jnp.float32),
    scratch_types=[
        pltpu.VMEM((NSETS * 128,), jnp.int32),
        pltpu.VMEM((2 * (B // NW),), jnp.int32),
        pltpu.VMEM((NSETS * CB * T,), jnp.float32),
        pltpu.VMEM((NSETS * CB * T, D), jnp.float32),
        pltpu.VMEM((CB, D), jnp.float32),
        pltpu.VMEM((CB * D,), jnp.float32),
        pltpu.VMEM((CB * D,), jnp.float32),
        pltpu.VMEM((16,), jnp.float32),
        pltpu.SemaphoreType.DMA,
    ],
    **_SC_PARAMS,
)
def _finisher(emb, tix, zb, uhm, ihm, items2, out,
              idx_v, iidx_v, zv, rows_v, gv, uv, iv, svec_ref, sem):
    w = _wid()
    bpw = B // NW
    pltpu.sync_copy(items2.at[pl.ds(w * 2 * bpw, 2 * bpw)], iidx_v)

    def chunk(ch, svec):
        b0 = w * bpw + ch * CB
        for s in range(NSETS):
            pltpu.sync_copy(tix.at[pl.ds(s * B * T + b0 * T, CB * T)],
                            idx_v.at[pl.ds(s * 128, 128)])
            pltpu.sync_copy(zb.at[pl.ds(s * B * T + b0 * T, CB * T)],
                            zv.at[pl.ds(s * CB * T, CB * T)])
        pltpu.sync_copy(uhm.at[pl.ds(b0 * D, CB * D)], uv)
        pltpu.sync_copy(ihm.at[pl.ds(b0 * D, CB * D)], iv)
        cps = [
            pltpu.async_copy(emb.at[idx_v.at[pl.ds(128 * k, 128)]],
                             rows_v.at[pl.ds(128 * k, 128)], sem)
            for k in range(NSETS)
        ]
        cps.append(pltpu.async_copy(emb.at[iidx_v.at[pl.ds(ch * 2 * CB, CB)]],
                                    gv, sem))
        for c in cps:
            c.wait()
        for bl in range(CB):
            eu = [uv[pl.ds(bl * D + 16 * k, 16)] for k in range(4)]
            ev = [iv[pl.ds(bl * D + 16 * k, 16)] + gv[bl, pl.ds(16 * k, 16)]
                  for k in range(4)]
            for s in range(NSETS):
                zo = s * CB * T + bl * T
                v0 = zv[pl.ds(zo, 16)]
                v1 = zv[pl.ds(zo + 16, 16)]
                e0 = jnp.exp(1.0 / (1.0 + jnp.exp(-v0)))
                e1 = jnp.exp(1.0 / (1.0 + jnp.exp(-v1)))
                tot = jnp.sum(e0 + e1)
                w0 = e0 / tot
                w1 = e1 / tot
                acc = [jnp.zeros((16,), jnp.float32) for _ in range(4)]
                for t in range(T):
                    wt = w0[t] if t < 16 else w1[t - 16]
                    for k in range(4):
                        acc[k] = acc[k] + wt * rows_v[zo + t, pl.ds(16 * k, 16)]
                if s < 2:
                    eu = [eu[k] + acc[k] for k in range(4)]
                else:
                    ev = [ev[k] + acc[k] for k in range(4)]
            dotv = eu[0] * ev[0]
            for k in range(1, 4):
                dotv = dotv + eu[k] * ev[k]
            dot = jnp.sum(dotv)
            scv = 1.0 / (1.0 + jnp.exp(-jnp.full((16,), dot, jnp.float32)))
            lane = (ch % 4) * CB + bl
            svec = jnp.where(lax.iota(jnp.int32, 16) == lane, scv, svec)
        svec_ref[...] = svec
        pltpu.sync_copy(svec_ref,
                        out.at[pl.ds(w * bpw + (ch // 4) * 16, 16)])
        return svec

    lax.fori_loop(0, NCH, chunk, jnp.zeros((16,), jnp.float32))


def kernel(items, user_h_0, user_r_0, user_t_0, user_h_1, user_r_1, user_t_1,
           item_h_0, item_r_0, item_t_0, item_h_1, item_r_1, item_t_1,
           entity_emb, relation_emb, W1, W2, W3):
    r1t = _r1_call(relation_emb, W1)

    h_idx = jnp.concatenate([
        user_h_0.T.reshape(-1), user_h_1.T.reshape(-1),
        item_h_0.T.reshape(-1), item_h_1.T.reshape(-1),
    ]).astype(jnp.int32)
    r_idx = jnp.concatenate([
        user_r_0.T.reshape(-1), user_r_1.T.reshape(-1),
        item_r_0.T.reshape(-1), item_r_1.T.reshape(-1),
    ]).astype(jnp.int32)
    gh, gr = _gather_rows(entity_emb, r1t, h_idx, r_idx)

    z, uhm, ihm = _att_call(gh, gr, W1, W2, W3)

    zb = z.reshape(NSETS, T, B).transpose(0, 2, 1).reshape(NSETS * B * T)
    tix = jnp.stack([user_t_0, user_t_1, item_t_0, item_t_1]
                    ).astype(jnp.int32).reshape(NSETS * B * T)
    items2 = jnp.pad(items.astype(jnp.int32).reshape(B // CB, CB),
                     ((0, 0), (0, CB))).reshape(2 * B)
    return _finisher(entity_emb, tix, zb, uhm.reshape(B * D),
                     ihm.reshape(B * D), items2)


# speedup vs baseline: 3.8533x; 1.3142x over previous
import functools

import jax
import jax.numpy as jnp
from jax import lax
from jax.experimental import pallas as pl
from jax.experimental.pallas import tpu as pltpu
from jax.experimental.pallas import tpu_sc as plsc

NC, NS = 2, 16
NW = NC * NS
B, T, D, NREL = 4096, 32, 64, 200
NSETS = 4
N = NSETS * T * B
RPW = N // NW
GCH = 4
GIT = RPW // (GCH * 128)

BB = 1024
NB = B // BB

CB = 4
NCH = (B // NW) // CB

_SC_PARAMS = dict(
    mesh=plsc.VectorSubcoreMesh(core_axis_name="c", subcore_axis_name="s"),
    compiler_params=pltpu.CompilerParams(
        use_tc_tiling_on_sc=False, needs_layout_passes=False),
)


def _wid():
    return lax.axis_index("s") * NC + lax.axis_index("c")


def _r1_body(rel_ref, w1_ref, out_ref):
    out_ref[...] = jnp.dot(rel_ref[...], w1_ref[D:2 * D, :],
                           preferred_element_type=jnp.float32,
                           precision=lax.Precision.HIGHEST)


def _r1_call(rel, w1):
    return pl.pallas_call(
        _r1_body,
        out_shape=jax.ShapeDtypeStruct((NREL, D), jnp.float32),
    )(rel, w1)


@functools.partial(
    pl.kernel,
    out_type=(jax.ShapeDtypeStruct((N, D), jnp.float32),
              jax.ShapeDtypeStruct((N, D), jnp.float32)),
    scratch_types=[
        pltpu.VMEM((GCH * 128,), jnp.int32),
        pltpu.VMEM((GCH * 128,), jnp.int32),
        pltpu.VMEM((GCH * 128, D), jnp.float32),
        pltpu.VMEM((GCH * 128, D), jnp.float32),
        pltpu.SemaphoreType.DMA,
    ],
    **_SC_PARAMS,
)
def _gather_rows(emb, r1t, hidx, ridx, outh, outr, hx_v, rx_v, hr_v, rr_v, sem):
    w = _wid()
    base = w * RPW

    def step(g, carry):
        r0 = base + GCH * 128 * g
        pltpu.sync_copy(hidx.at[pl.ds(r0, GCH * 128)], hx_v)
        pltpu.sync_copy(ridx.at[pl.ds(r0, GCH * 128)], rx_v)
        cps = []
        for k in range(GCH):
            cps.append(pltpu.async_copy(emb.at[hx_v.at[pl.ds(128 * k, 128)]],
                                        hr_v.at[pl.ds(128 * k, 128)], sem))
            cps.append(pltpu.async_copy(r1t.at[rx_v.at[pl.ds(128 * k, 128)]],
                                        rr_v.at[pl.ds(128 * k, 128)], sem))
        for c in cps:
            c.wait()
        pltpu.sync_copy(hr_v, outh.at[pl.ds(r0, GCH * 128)])
        pltpu.sync_copy(rr_v, outr.at[pl.ds(r0, GCH * 128)])
        return carry

    lax.fori_loop(0, GIT, step, 0)


def _att_body(gh_ref, gr_ref, w1d_ref, w2d_ref, w3d_ref,
              z_ref, uhm_ref, ihm_ref):
    j = pl.program_id(1)

    h = gh_ref[...]
    x = jnp.dot(h.astype(jnp.bfloat16), w1d_ref[...],
                preferred_element_type=jnp.float32)
    x = jnp.maximum(x + gr_ref[...], 0.0)
    x = jnp.maximum(jnp.dot(x.astype(jnp.bfloat16), w2d_ref[...],
                            preferred_element_type=jnp.float32), 0.0)
    zp = jnp.dot(x.astype(jnp.bfloat16), w3d_ref[...],
                 preferred_element_type=jnp.float32)
    iot = lax.broadcasted_iota(jnp.int32, (1, 4 * D), 1)
    acc = (zp[:, 0:1] * (iot == j).astype(jnp.float32)
           + zp[:, 1:2] * (iot == j + 2 * D).astype(jnp.float32))

    @pl.when(j == 0)
    def _():
        z_ref[...] = acc

    @pl.when(j > 0)
    def _():
        z_ref[...] = z_ref[...] + acc

    inv = 1.0 / T

    @pl.when(j == 0)
    def _():
        uhm_ref[...] = h * inv

    @pl.when((j >= 1) & (j < T))
    def _():
        uhm_ref[...] = uhm_ref[...] + h * inv

    @pl.when(j == 2 * T)
    def _():
        ihm_ref[...] = h * inv

    @pl.when((j > 2 * T) & (j < 3 * T))
    def _():
        ihm_ref[...] = ihm_ref[...] + h * inv


def _att_call(ghp, grp, w1d, w2d, w3d):
    bbp = BB // 2
    return pl.pallas_call(
        _att_body,
        grid=(NB, NSETS * T),
        in_specs=[
            pl.BlockSpec((bbp, 2 * D), lambda i, j: (j * NB + i, 0)),
            pl.BlockSpec((bbp, 2 * D), lambda i, j: (j * NB + i, 0)),
            pl.BlockSpec((2 * D, 2 * D), lambda i, j: (0, 0)),
            pl.BlockSpec((2 * D, 2 * D), lambda i, j: (0, 0)),
            pl.BlockSpec((2 * D, 2), lambda i, j: (0, 0)),
        ],
        out_specs=[
            pl.BlockSpec((bbp, 4 * D), lambda i, j: (i, 0)),
            pl.BlockSpec((bbp, 2 * D), lambda i, j: (i, 0)),
            pl.BlockSpec((bbp, 2 * D), lambda i, j: (i, 0)),
        ],
        out_shape=[
            jax.ShapeDtypeStruct((B // 2, 4 * D), jnp.float32),
            jax.ShapeDtypeStruct((B // 2, 2 * D), jnp.float32),
            jax.ShapeDtypeStruct((B // 2, 2 * D), jnp.float32),
        ],
    )(ghp, grp, w1d, w2d, w3d)


@functools.partial(
    pl.kernel,
    out_type=jax.ShapeDtypeStruct((B,), jnp.float32),
    scratch_types=[
        pltpu.VMEM((NSETS * 128,), jnp.int32),
        pltpu.VMEM((2 * (B // NW),), jnp.int32),
        pltpu.VMEM((NSETS * CB * T,), jnp.float32),
        pltpu.VMEM((NSETS * CB * T, D), jnp.float32),
        pltpu.VMEM((CB, D), jnp.float32),
        pltpu.VMEM((CB * D,), jnp.float32),
        pltpu.VMEM((CB * D,), jnp.float32),
        pltpu.VMEM((16,), jnp.float32),
        pltpu.SemaphoreType.DMA,
    ],
    **_SC_PARAMS,
)
def _finisher(emb, tix, zb, uhm, ihm, items2, out,
              idx_v, iidx_v, zv, rows_v, gv, uv, iv, svec_ref, sem):
    w = _wid()
    bpw = B // NW
    pltpu.sync_copy(items2.at[pl.ds(w * 2 * bpw, 2 * bpw)], iidx_v)

    def chunk(ch, svec):
        b0 = w * bpw + ch * CB
        for s in range(NSETS):
            pltpu.sync_copy(tix.at[pl.ds(s * B * T + b0 * T, CB * T)],
                            idx_v.at[pl.ds(s * 128, 128)])
        pltpu.sync_copy(zb.at[pl.ds(b0 * 128, CB * 128)], zv)
        pltpu.sync_copy(uhm.at[pl.ds(b0 * D, CB * D)], uv)
        pltpu.sync_copy(ihm.at[pl.ds(b0 * D, CB * D)], iv)
        cps = [
            pltpu.async_copy(emb.at[idx_v.at[pl.ds(128 * k, 128)]],
                             rows_v.at[pl.ds(128 * k, 128)], sem)
            for k in range(NSETS)
        ]
        cps.append(pltpu.async_copy(emb.at[iidx_v.at[pl.ds(ch * 2 * CB, CB)]],
                                    gv, sem))
        for c in cps:
            c.wait()
        for bl in range(CB):
            eu = [uv[pl.ds(bl * D + 16 * k, 16)] for k in range(4)]
            ev = [iv[pl.ds(bl * D + 16 * k, 16)] + gv[bl, pl.ds(16 * k, 16)]
                  for k in range(4)]
            for s in range(NSETS):
                zo = bl * 128 + s * T
                rb = s * CB * T + bl * T
                v0 = zv[pl.ds(zo, 16)]
                v1 = zv[pl.ds(zo + 16, 16)]
                e0 = jnp.exp(1.0 / (1.0 + jnp.exp(-v0)))
                e1 = jnp.exp(1.0 / (1.0 + jnp.exp(-v1)))
                tot = jnp.sum(e0 + e1)
                w0 = e0 / tot
                w1 = e1 / tot
                acc = [jnp.zeros((16,), jnp.float32) for _ in range(4)]
                for t in range(T):
                    wt = w0[t] if t < 16 else w1[t - 16]
                    for k in range(4):
                        acc[k] = acc[k] + wt * rows_v[rb + t, pl.ds(16 * k, 16)]
                if s < 2:
                    eu = [eu[k] + acc[k] for k in range(4)]
                else:
                    ev = [ev[k] + acc[k] for k in range(4)]
            dotv = eu[0] * ev[0]
            for k in range(1, 4):
                dotv = dotv + eu[k] * ev[k]
            dot = jnp.sum(dotv)
            scv = 1.0 / (1.0 + jnp.exp(-jnp.full((16,), dot, jnp.float32)))
            lane = (ch % 4) * CB + bl
            svec = jnp.where(lax.iota(jnp.int32, 16) == lane, scv, svec)
        svec_ref[...] = svec
        pltpu.sync_copy(svec_ref,
                        out.at[pl.ds(w * bpw + (ch // 4) * 16, 16)])
        return svec

    lax.fori_loop(0, NCH, chunk, jnp.zeros((16,), jnp.float32))


def kernel(items, user_h_0, user_r_0, user_t_0, user_h_1, user_r_1, user_t_1,
           item_h_0, item_r_0, item_t_0, item_h_1, item_r_1, item_t_1,
           entity_emb, relation_emb, W1, W2, W3):
    r1t = _r1_call(relation_emb, W1)

    h_idx = jnp.concatenate([
        user_h_0.T.reshape(-1), user_h_1.T.reshape(-1),
        item_h_0.T.reshape(-1), item_h_1.T.reshape(-1),
    ]).astype(jnp.int32)
    r_idx = jnp.concatenate([
        user_r_0.T.reshape(-1), user_r_1.T.reshape(-1),
        item_r_0.T.reshape(-1), item_r_1.T.reshape(-1),
    ]).astype(jnp.int32)
    gh, gr = _gather_rows(entity_emb, r1t, h_idx, r_idx)

    zero = jnp.zeros((D, D), jnp.bfloat16)
    w1a = W1[:D, :].astype(jnp.bfloat16)
    w1d = jnp.block([[w1a, zero], [zero, w1a]])
    w2b = W2.astype(jnp.bfloat16)
    w2d = jnp.block([[w2b, zero], [zero, w2b]])
    zc = jnp.zeros((D, 1), jnp.bfloat16)
    w3b = W3.astype(jnp.bfloat16)
    w3d = jnp.block([[w3b, zc], [zc, w3b]])
    z, uhm, ihm = _att_call(gh.reshape(N // 2, 2 * D),
                            gr.reshape(N // 2, 2 * D), w1d, w2d, w3d)

    tix = jnp.stack([user_t_0, user_t_1, item_t_0, item_t_1]
                    ).astype(jnp.int32).reshape(NSETS * B * T)
    items2 = jnp.pad(items.astype(jnp.int32).reshape(B // CB, CB),
                     ((0, 0), (0, CB))).reshape(2 * B)
    return _finisher(entity_emb, tix, z.reshape(B * 2 * D),
                     uhm.reshape(B * D), ihm.reshape(B * D), items2)


# speedup vs baseline: 4.6308x; 1.2018x over previous
import functools

import jax
import jax.numpy as jnp
from jax import lax
from jax.experimental import pallas as pl
from jax.experimental.pallas import tpu as pltpu
from jax.experimental.pallas import tpu_sc as plsc

NC, NS = 2, 16
NW = NC * NS
B, T, D, NREL = 4096, 32, 64, 200
NSETS = 4
N = NSETS * T * B
RPW = N // NW
GCH = 4
GIT = RPW // (GCH * 128)

BB = 1024
NB = B // BB

CB = 4
NCH = (B // NW) // CB

_SC_PARAMS = dict(
    mesh=plsc.VectorSubcoreMesh(core_axis_name="c", subcore_axis_name="s"),
    compiler_params=pltpu.CompilerParams(
        use_tc_tiling_on_sc=False, needs_layout_passes=False),
)


def _wid():
    return lax.axis_index("s") * NC + lax.axis_index("c")


NENT = 1000000
EB = 8192
NPKB = (NENT + EB - 1) // EB


def _repack_body(et_ref, out_ref):
    out_ref[:, 0:D] = jnp.transpose(et_ref[:, 0:EB // 2])
    out_ref[:, D:2 * D] = jnp.transpose(et_ref[:, EB // 2:EB])


def _repack_call(et):
    return pl.pallas_call(
        _repack_body,
        grid=(NPKB,),
        in_specs=[pl.BlockSpec((D, EB), lambda i: (0, i))],
        out_specs=pl.BlockSpec((EB // 2, 2 * D), lambda i: (i, 0)),
        out_shape=jax.ShapeDtypeStruct((NPKB * (EB // 2), 2 * D), jnp.float32),
    )(et)


def _pidx(e):
    g = e // EB
    l = e % EB
    return 2 * (g * (EB // 2) + l % (EB // 2)) + l // (EB // 2)


def _r1_body(rel_ref, w1_ref, out_ref):
    out_ref[...] = jnp.dot(rel_ref[...], w1_ref[D:2 * D, :],
                           preferred_element_type=jnp.float32,
                           precision=lax.Precision.HIGHEST)


def _r1_call(rel, w1):
    return pl.pallas_call(
        _r1_body,
        out_shape=jax.ShapeDtypeStruct((NREL, D), jnp.float32),
    )(rel, w1)


@functools.partial(
    pl.kernel,
    out_type=(jax.ShapeDtypeStruct((N, D), jnp.float32),
              jax.ShapeDtypeStruct((N, D), jnp.float32)),
    scratch_types=[
        pltpu.VMEM((GCH * 128,), jnp.int32),
        pltpu.VMEM((GCH * 128,), jnp.int32),
        pltpu.VMEM((GCH * 128, D), jnp.float32),
        pltpu.VMEM((GCH * 128, D), jnp.float32),
        pltpu.SemaphoreType.DMA,
    ],
    **_SC_PARAMS,
)
def _gather_rows(emb, r1t, hidx, ridx, outh, outr, hx_v, rx_v, hr_v, rr_v, sem):
    w = _wid()
    base = w * RPW

    def step(g, carry):
        r0 = base + GCH * 128 * g
        pltpu.sync_copy(hidx.at[pl.ds(r0, GCH * 128)], hx_v)
        pltpu.sync_copy(ridx.at[pl.ds(r0, GCH * 128)], rx_v)
        cps = []
        for k in range(GCH):
            cps.append(pltpu.async_copy(emb.at[hx_v.at[pl.ds(128 * k, 128)]],
                                        hr_v.at[pl.ds(128 * k, 128)], sem))
            cps.append(pltpu.async_copy(r1t.at[rx_v.at[pl.ds(128 * k, 128)]],
                                        rr_v.at[pl.ds(128 * k, 128)], sem))
        for c in cps:
            c.wait()
        pltpu.sync_copy(hr_v, outh.at[pl.ds(r0, GCH * 128)])
        pltpu.sync_copy(rr_v, outr.at[pl.ds(r0, GCH * 128)])
        return carry

    lax.fori_loop(0, GIT, step, 0)


def _att_body(gh_ref, gr_ref, w1d_ref, w2d_ref, w3d_ref,
              z_ref, uhm_ref, ihm_ref):
    j = pl.program_id(1)

    h = gh_ref[...]
    x = jnp.dot(h.astype(jnp.bfloat16), w1d_ref[...],
                preferred_element_type=jnp.float32)
    x = jnp.maximum(x + gr_ref[...], 0.0)
    x = jnp.maximum(jnp.dot(x.astype(jnp.bfloat16), w2d_ref[...],
                            preferred_element_type=jnp.float32), 0.0)
    zp = jnp.dot(x.astype(jnp.bfloat16), w3d_ref[...],
                 preferred_element_type=jnp.float32)
    iot = lax.broadcasted_iota(jnp.int32, (1, 4 * D), 1)
    acc = (zp[:, 0:1] * (iot == j).astype(jnp.float32)
           + zp[:, 1:2] * (iot == j + 2 * D).astype(jnp.float32))

    @pl.when(j == 0)
    def _():
        z_ref[...] = acc

    @pl.when(j > 0)
    def _():
        z_ref[...] = z_ref[...] + acc

    inv = 1.0 / T

    @pl.when(j == 0)
    def _():
        uhm_ref[...] = h * inv

    @pl.when((j >= 1) & (j < T))
    def _():
        uhm_ref[...] = uhm_ref[...] + h * inv

    @pl.when(j == 2 * T)
    def _():
        ihm_ref[...] = h * inv

    @pl.when((j > 2 * T) & (j < 3 * T))
    def _():
        ihm_ref[...] = ihm_ref[...] + h * inv


def _att_call(ghp, grp, w1d, w2d, w3d):
    bbp = BB // 2
    return pl.pallas_call(
        _att_body,
        grid=(NB, NSETS * T),
        in_specs=[
            pl.BlockSpec((bbp, 2 * D), lambda i, j: (j * NB + i, 0)),
            pl.BlockSpec((bbp, 2 * D), lambda i, j: (j * NB + i, 0)),
            pl.BlockSpec((2 * D, 2 * D), lambda i, j: (0, 0)),
            pl.BlockSpec((2 * D, 2 * D), lambda i, j: (0, 0)),
            pl.BlockSpec((2 * D, 2), lambda i, j: (0, 0)),
        ],
        out_specs=[
            pl.BlockSpec((bbp, 4 * D), lambda i, j: (i, 0)),
            pl.BlockSpec((bbp, 2 * D), lambda i, j: (i, 0)),
            pl.BlockSpec((bbp, 2 * D), lambda i, j: (i, 0)),
        ],
        out_shape=[
            jax.ShapeDtypeStruct((B // 2, 4 * D), jnp.float32),
            jax.ShapeDtypeStruct((B // 2, 2 * D), jnp.float32),
            jax.ShapeDtypeStruct((B // 2, 2 * D), jnp.float32),
        ],
    )(ghp, grp, w1d, w2d, w3d)


@functools.partial(
    pl.kernel,
    out_type=jax.ShapeDtypeStruct((B,), jnp.float32),
    scratch_types=[
        pltpu.VMEM((NSETS * 128,), jnp.int32),
        pltpu.VMEM((2 * (B // NW),), jnp.int32),
        pltpu.VMEM((NSETS * CB * T,), jnp.float32),
        pltpu.VMEM((NSETS * CB * T, D), jnp.float32),
        pltpu.VMEM((CB, D), jnp.float32),
        pltpu.VMEM((CB * D,), jnp.float32),
        pltpu.VMEM((CB * D,), jnp.float32),
        pltpu.VMEM((16,), jnp.float32),
        pltpu.SemaphoreType.DMA,
    ],
    **_SC_PARAMS,
)
def _finisher(emb, tix, zb, uhm, ihm, items2, out,
              idx_v, iidx_v, zv, rows_v, gv, uv, iv, svec_ref, sem):
    w = _wid()
    bpw = B // NW
    pltpu.sync_copy(items2.at[pl.ds(w * 2 * bpw, 2 * bpw)], iidx_v)

    def chunk(ch, svec):
        b0 = w * bpw + ch * CB
        for s in range(NSETS):
            pltpu.sync_copy(tix.at[pl.ds(s * B * T + b0 * T, CB * T)],
                            idx_v.at[pl.ds(s * 128, 128)])
        pltpu.sync_copy(zb.at[pl.ds(b0 * 128, CB * 128)], zv)
        pltpu.sync_copy(uhm.at[pl.ds(b0 * D, CB * D)], uv)
        pltpu.sync_copy(ihm.at[pl.ds(b0 * D, CB * D)], iv)
        cps = [
            pltpu.async_copy(emb.at[idx_v.at[pl.ds(128 * k, 128)]],
                             rows_v.at[pl.ds(128 * k, 128)], sem)
            for k in range(NSETS)
        ]
        cps.append(pltpu.async_copy(emb.at[iidx_v.at[pl.ds(ch * 2 * CB, CB)]],
                                    gv, sem))
        for c in cps:
            c.wait()
        for bl in range(CB):
            eu = [uv[pl.ds(bl * D + 16 * k, 16)] for k in range(4)]
            ev = [iv[pl.ds(bl * D + 16 * k, 16)] + gv[bl, pl.ds(16 * k, 16)]
                  for k in range(4)]
            for s in range(NSETS):
                zo = bl * 128 + s * T
                rb = s * CB * T + bl * T
                v0 = zv[pl.ds(zo, 16)]
                v1 = zv[pl.ds(zo + 16, 16)]
                e0 = jnp.exp(1.0 / (1.0 + jnp.exp(-v0)))
                e1 = jnp.exp(1.0 / (1.0 + jnp.exp(-v1)))
                tot = jnp.sum(e0 + e1)
                w0 = e0 / tot
                w1 = e1 / tot
                acc = [jnp.zeros((16,), jnp.float32) for _ in range(4)]
                for t in range(T):
                    wt = w0[t] if t < 16 else w1[t - 16]
                    for k in range(4):
                        acc[k] = acc[k] + wt * rows_v[rb + t, pl.ds(16 * k, 16)]
                if s < 2:
                    eu = [eu[k] + acc[k] for k in range(4)]
                else:
                    ev = [ev[k] + acc[k] for k in range(4)]
            dotv = eu[0] * ev[0]
            for k in range(1, 4):
                dotv = dotv + eu[k] * ev[k]
            dot = jnp.sum(dotv)
            scv = 1.0 / (1.0 + jnp.exp(-jnp.full((16,), dot, jnp.float32)))
            lane = (ch % 4) * CB + bl
            svec = jnp.where(lax.iota(jnp.int32, 16) == lane, scv, svec)
        svec_ref[...] = svec
        pltpu.sync_copy(svec_ref,
                        out.at[pl.ds(w * bpw + (ch // 4) * 16, 16)])
        return svec

    lax.fori_loop(0, NCH, chunk, jnp.zeros((16,), jnp.float32))


def kernel(items, user_h_0, user_r_0, user_t_0, user_h_1, user_r_1, user_t_1,
           item_h_0, item_r_0, item_t_0, item_h_1, item_r_1, item_t_1,
           entity_emb, relation_emb, W1, W2, W3):
    r1t = _r1_call(relation_emb, W1)
    emb_lin = _repack_call(entity_emb.T).reshape(NPKB * EB, D)

    h_idx = _pidx(jnp.concatenate([
        user_h_0.T.reshape(-1), user_h_1.T.reshape(-1),
        item_h_0.T.reshape(-1), item_h_1.T.reshape(-1),
    ]).astype(jnp.int32))
    r_idx = jnp.concatenate([
        user_r_0.T.reshape(-1), user_r_1.T.reshape(-1),
        item_r_0.T.reshape(-1), item_r_1.T.reshape(-1),
    ]).astype(jnp.int32)
    gh, gr = _gather_rows(emb_lin, r1t, h_idx, r_idx)

    zero = jnp.zeros((D, D), jnp.bfloat16)
    w1a = W1[:D, :].astype(jnp.bfloat16)
    w1d = jnp.block([[w1a, zero], [zero, w1a]])
    w2b = W2.astype(jnp.bfloat16)
    w2d = jnp.block([[w2b, zero], [zero, w2b]])
    zc = jnp.zeros((D, 1), jnp.bfloat16)
    w3b = W3.astype(jnp.bfloat16)
    w3d = jnp.block([[w3b, zc], [zc, w3b]])
    z, uhm, ihm = _att_call(gh.reshape(N // 2, 2 * D),
                            gr.reshape(N // 2, 2 * D), w1d, w2d, w3d)

    tix = _pidx(jnp.stack([user_t_0, user_t_1, item_t_0, item_t_1]
                          ).astype(jnp.int32).reshape(NSETS * B * T))
    items2 = jnp.pad(_pidx(items.astype(jnp.int32)).reshape(B // CB, CB),
                     ((0, 0), (0, CB))).reshape(2 * B)
    return _finisher(emb_lin, tix, z.reshape(B * 2 * D),
                     uhm.reshape(B * D), ihm.reshape(B * D), items2)


# speedup vs baseline: 4.7590x; 1.0277x over previous
import functools

import jax
import jax.numpy as jnp
from jax import lax
from jax.experimental import pallas as pl
from jax.experimental.pallas import tpu as pltpu
from jax.experimental.pallas import tpu_sc as plsc

NC, NS = 2, 16
NW = NC * NS
B, T, D, NREL = 4096, 32, 64, 200
NSETS = 4
N = NSETS * T * B
RPW = N // NW
GCH = 4
GIT = RPW // (GCH * 128)

BB = 1024
NB = B // BB

CB = 4
NCH = (B // NW) // CB

_SC_PARAMS = dict(
    mesh=plsc.VectorSubcoreMesh(core_axis_name="c", subcore_axis_name="s"),
    compiler_params=pltpu.CompilerParams(
        use_tc_tiling_on_sc=False, needs_layout_passes=False),
)


def _wid():
    return lax.axis_index("s") * NC + lax.axis_index("c")


NENT = 1000000
EB = 8192
NPKB = (NENT + EB - 1) // EB


def _repack_body(et_ref, out_ref):
    out_ref[:, 0:D] = jnp.transpose(et_ref[:, 0:EB // 2])
    out_ref[:, D:2 * D] = jnp.transpose(et_ref[:, EB // 2:EB])


def _repack_call(et):
    return pl.pallas_call(
        _repack_body,
        grid=(NPKB,),
        in_specs=[pl.BlockSpec((D, EB), lambda i: (0, i))],
        out_specs=pl.BlockSpec((EB // 2, 2 * D), lambda i: (i, 0)),
        out_shape=jax.ShapeDtypeStruct((NPKB * (EB // 2), 2 * D), jnp.float32),
    )(et)


def _pidx(e):
    g = e // EB
    l = e % EB
    return 2 * (g * (EB // 2) + l % (EB // 2)) + l // (EB // 2)


def _r1_body(rel_ref, w1_ref, out_ref):
    out_ref[...] = jnp.dot(rel_ref[...], w1_ref[D:2 * D, :],
                           preferred_element_type=jnp.float32,
                           precision=lax.Precision.HIGHEST)


def _r1_call(rel, w1):
    return pl.pallas_call(
        _r1_body,
        out_shape=jax.ShapeDtypeStruct((NREL, D), jnp.float32),
    )(rel, w1)


CHN = GCH * 128


@functools.partial(
    pl.kernel,
    out_type=jax.ShapeDtypeStruct((N, D), jnp.float32),
    scratch_types=[
        pltpu.VMEM((CHN,), jnp.int32),
        pltpu.VMEM((CHN,), jnp.int32),
        pltpu.VMEM((CHN, D), jnp.float32),
        pltpu.VMEM((CHN, D), jnp.float32),
        pltpu.SemaphoreType.DMA,
        pltpu.SemaphoreType.DMA,
        pltpu.SemaphoreType.DMA,
    ],
    **_SC_PARAMS,
)
def _gather_rows(tbl, idx, out, ix0, ix1, rv0, rv1, semg, semw0, semw1):
    w = _wid()
    base = w * RPW
    last = base + (GIT - 1) * CHN

    def fire(ixb, rvb):
        return [pltpu.async_copy(tbl.at[ixb.at[pl.ds(128 * k, 128)]],
                                 rvb.at[pl.ds(128 * k, 128)], semg)
                for k in range(GCH)]

    pltpu.sync_copy(idx.at[pl.ds(base, CHN)], ix0)
    c0 = fire(ix0, rv0)
    pltpu.sync_copy(idx.at[pl.ds(base + CHN, CHN)], ix1)
    for c in c0:
        c.wait()
    pltpu.async_copy(rv0, out.at[pl.ds(base, CHN)], semw0)
    c1 = fire(ix1, rv1)
    pltpu.sync_copy(idx.at[pl.ds(base + 2 * CHN, CHN)], ix0)
    for c in c1:
        c.wait()
    pltpu.async_copy(rv1, out.at[pl.ds(base + CHN, CHN)], semw1)

    def step(m, carry):
        g0 = base + 2 * m * CHN
        g1 = g0 + CHN
        pltpu.make_async_copy(rv0, out.at[pl.ds(g0, CHN)], semw0).wait()
        ca = fire(ix0, rv0)
        pltpu.sync_copy(idx.at[pl.ds(g1, CHN)], ix1)
        for c in ca:
            c.wait()
        pltpu.async_copy(rv0, out.at[pl.ds(g0, CHN)], semw0)
        pltpu.make_async_copy(rv1, out.at[pl.ds(g1, CHN)], semw1).wait()
        cb = fire(ix1, rv1)
        nxt2 = jnp.minimum(g1 + CHN, last)
        pltpu.sync_copy(idx.at[pl.ds(nxt2, CHN)], ix0)
        for c in cb:
            c.wait()
        pltpu.async_copy(rv1, out.at[pl.ds(g1, CHN)], semw1)
        return carry

    lax.fori_loop(1, GIT // 2, step, 0)
    pltpu.make_async_copy(rv0, out.at[pl.ds(last - CHN, CHN)], semw0).wait()
    pltpu.make_async_copy(rv1, out.at[pl.ds(last, CHN)], semw1).wait()


def _att_body(gh_ref, gr_ref, w1d_ref, w2d_ref, w3d_ref,
              z_ref, uhm_ref, ihm_ref):
    j = pl.program_id(1)

    h = gh_ref[...]
    x = jnp.dot(h.astype(jnp.bfloat16), w1d_ref[...],
                preferred_element_type=jnp.float32)
    x = jnp.maximum(x + gr_ref[...], 0.0)
    x = jnp.maximum(jnp.dot(x.astype(jnp.bfloat16), w2d_ref[...],
                            preferred_element_type=jnp.float32), 0.0)
    zp = jnp.dot(x.astype(jnp.bfloat16), w3d_ref[...],
                 preferred_element_type=jnp.float32)
    iot = lax.broadcasted_iota(jnp.int32, (1, 4 * D), 1)
    acc = (zp[:, 0:1] * (iot == j).astype(jnp.float32)
           + zp[:, 1:2] * (iot == j + 2 * D).astype(jnp.float32))

    @pl.when(j == 0)
    def _():
        z_ref[...] = acc

    @pl.when(j > 0)
    def _():
        z_ref[...] = z_ref[...] + acc

    inv = 1.0 / T

    @pl.when(j == 0)
    def _():
        uhm_ref[...] = h * inv

    @pl.when((j >= 1) & (j < T))
    def _():
        uhm_ref[...] = uhm_ref[...] + h * inv

    @pl.when(j == 2 * T)
    def _():
        ihm_ref[...] = h * inv

    @pl.when((j > 2 * T) & (j < 3 * T))
    def _():
        ihm_ref[...] = ihm_ref[...] + h * inv


def _att_call(ghp, grp, w1d, w2d, w3d):
    bbp = BB // 2
    return pl.pallas_call(
        _att_body,
        grid=(NB, NSETS * T),
        in_specs=[
            pl.BlockSpec((bbp, 2 * D), lambda i, j: (j * NB + i, 0)),
            pl.BlockSpec((bbp, 2 * D), lambda i, j: (j * NB + i, 0)),
            pl.BlockSpec((2 * D, 2 * D), lambda i, j: (0, 0)),
            pl.BlockSpec((2 * D, 2 * D), lambda i, j: (0, 0)),
            pl.BlockSpec((2 * D, 2), lambda i, j: (0, 0)),
        ],
        out_specs=[
            pl.BlockSpec((bbp, 4 * D), lambda i, j: (i, 0)),
            pl.BlockSpec((bbp, 2 * D), lambda i, j: (i, 0)),
            pl.BlockSpec((bbp, 2 * D), lambda i, j: (i, 0)),
        ],
        out_shape=[
            jax.ShapeDtypeStruct((B // 2, 4 * D), jnp.float32),
            jax.ShapeDtypeStruct((B // 2, 2 * D), jnp.float32),
            jax.ShapeDtypeStruct((B // 2, 2 * D), jnp.float32),
        ],
    )(ghp, grp, w1d, w2d, w3d)


@functools.partial(
    pl.kernel,
    out_type=jax.ShapeDtypeStruct((B,), jnp.float32),
    scratch_types=[
        pltpu.VMEM((NSETS * 128,), jnp.int32),
        pltpu.VMEM((2 * (B // NW),), jnp.int32),
        pltpu.VMEM((NSETS * CB * T,), jnp.float32),
        pltpu.VMEM((NSETS * CB * T, D), jnp.float32),
        pltpu.VMEM((CB, D), jnp.float32),
        pltpu.VMEM((CB * D,), jnp.float32),
        pltpu.VMEM((CB * D,), jnp.float32),
        pltpu.VMEM((16,), jnp.float32),
        pltpu.SemaphoreType.DMA,
    ],
    **_SC_PARAMS,
)
def _finisher(emb, tix, zb, uhm, ihm, items2, out,
              idx_v, iidx_v, zv, rows_v, gv, uv, iv, svec_ref, sem):
    w = _wid()
    bpw = B // NW
    pltpu.sync_copy(items2.at[pl.ds(w * 2 * bpw, 2 * bpw)], iidx_v)

    def chunk(ch, svec):
        b0 = w * bpw + ch * CB
        for s in range(NSETS):
            pltpu.sync_copy(tix.at[pl.ds(s * B * T + b0 * T, CB * T)],
                            idx_v.at[pl.ds(s * 128, 128)])
        pltpu.sync_copy(zb.at[pl.ds(b0 * 128, CB * 128)], zv)
        pltpu.sync_copy(uhm.at[pl.ds(b0 * D, CB * D)], uv)
        pltpu.sync_copy(ihm.at[pl.ds(b0 * D, CB * D)], iv)
        cps = [
            pltpu.async_copy(emb.at[idx_v.at[pl.ds(128 * k, 128)]],
                             rows_v.at[pl.ds(128 * k, 128)], sem)
            for k in range(NSETS)
        ]
        cps.append(pltpu.async_copy(emb.at[iidx_v.at[pl.ds(ch * 2 * CB, CB)]],
                                    gv, sem))
        for c in cps:
            c.wait()
        for bl in range(CB):
            eu = [uv[pl.ds(bl * D + 16 * k, 16)] for k in range(4)]
            ev = [iv[pl.ds(bl * D + 16 * k, 16)] + gv[bl, pl.ds(16 * k, 16)]
                  for k in range(4)]
            for s in range(NSETS):
                zo = bl * 128 + s * T
                rb = s * CB * T + bl * T
                v0 = zv[pl.ds(zo, 16)]
                v1 = zv[pl.ds(zo + 16, 16)]
                e0 = jnp.exp(1.0 / (1.0 + jnp.exp(-v0)))
                e1 = jnp.exp(1.0 / (1.0 + jnp.exp(-v1)))
                tot = jnp.sum(e0 + e1)
                w0 = e0 / tot
                w1 = e1 / tot
                acc = [jnp.zeros((16,), jnp.float32) for _ in range(4)]
                for t in range(T):
                    wt = w0[t] if t < 16 else w1[t - 16]
                    for k in range(4):
                        acc[k] = acc[k] + wt * rows_v[rb + t, pl.ds(16 * k, 16)]
                if s < 2:
                    eu = [eu[k] + acc[k] for k in range(4)]
                else:
                    ev = [ev[k] + acc[k] for k in range(4)]
            dotv = eu[0] * ev[0]
            for k in range(1, 4):
                dotv = dotv + eu[k] * ev[k]
            dot = jnp.sum(dotv)
            scv = 1.0 / (1.0 + jnp.exp(-jnp.full((16,), dot, jnp.float32)))
            lane = (ch % 4) * CB + bl
            svec = jnp.where(lax.iota(jnp.int32, 16) == lane, scv, svec)
        svec_ref[...] = svec
        pltpu.sync_copy(svec_ref,
                        out.at[pl.ds(w * bpw + (ch // 4) * 16, 16)])
        return svec

    lax.fori_loop(0, NCH, chunk, jnp.zeros((16,), jnp.float32))


def kernel(items, user_h_0, user_r_0, user_t_0, user_h_1, user_r_1, user_t_1,
           item_h_0, item_r_0, item_t_0, item_h_1, item_r_1, item_t_1,
           entity_emb, relation_emb, W1, W2, W3):
    r1t = _r1_call(relation_emb, W1)
    emb_lin = _repack_call(entity_emb.T).reshape(NPKB * EB, D)

    h_idx = _pidx(jnp.concatenate([
        user_h_0.T.reshape(-1), user_h_1.T.reshape(-1),
        item_h_0.T.reshape(-1), item_h_1.T.reshape(-1),
    ]).astype(jnp.int32))
    r_idx = jnp.concatenate([
        user_r_0.T.reshape(-1), user_r_1.T.reshape(-1),
        item_r_0.T.reshape(-1), item_r_1.T.reshape(-1),
    ]).astype(jnp.int32)
    gr = _gather_rows(r1t, r_idx)
    gh = _gather_rows(emb_lin, h_idx)

    zero = jnp.zeros((D, D), jnp.bfloat16)
    w1a = W1[:D, :].astype(jnp.bfloat16)
    w1d = jnp.block([[w1a, zero], [zero, w1a]])
    w2b = W2.astype(jnp.bfloat16)
    w2d = jnp.block([[w2b, zero], [zero, w2b]])
    zc = jnp.zeros((D, 1), jnp.bfloat16)
    w3b = W3.astype(jnp.bfloat16)
    w3d = jnp.block([[w3b, zc], [zc, w3b]])
    z, uhm, ihm = _att_call(gh.reshape(N // 2, 2 * D),
                            gr.reshape(N // 2, 2 * D), w1d, w2d, w3d)

    tix = _pidx(jnp.stack([user_t_0, user_t_1, item_t_0, item_t_1]
                          ).astype(jnp.int32).reshape(NSETS * B * T))
    items2 = jnp.pad(_pidx(items.astype(jnp.int32)).reshape(B // CB, CB),
                     ((0, 0), (0, CB))).reshape(2 * B)
    return _finisher(emb_lin, tix, z.reshape(B * 2 * D),
                     uhm.reshape(B * D), ihm.reshape(B * D), items2)
